# baseline (speedup 1.0000x reference)
import functools

import jax
import jax.numpy as jnp
from jax import lax
from jax.experimental import pallas as pl
from jax.experimental.pallas import tpu as pltpu
from jax.experimental.pallas import tpu_sc as plsc

R = 32
P = 4
G = R // P
V = R * R * R
B = 16
C = 3
N = 131072
HIDDEN = 384
SUBL = 1024
LANE = 128
CH = 4096
NCHUNK = N // CH


def _stats_body(pc_ref, par_ref, mean_ref):
    p = pl.program_id(1)

    @pl.when(p == 0)
    def _():
        for ci in range(C):
            mean_ref[ci] = jnp.sum(pc_ref[0, ci]) / N

    @pl.when(p == 1)
    def _():
        n0 = pc_ref[0, 0] - mean_ref[0]
        n1 = pc_ref[0, 1] - mean_ref[1]
        n2 = pc_ref[0, 2] - mean_ref[2]
        s = n0 * n0 + n1 * n1 + n2 * n2
        md2 = jnp.sqrt(jnp.max(s)) * 2.0
        a = float(R) / md2
        li = jax.lax.broadcasted_iota(jnp.int32, (1, LANE), 1)
        row = jnp.where(li == 3, a, 0.0)
        for ci in range(C):
            bc = float(R) * 0.5 - mean_ref[ci] * a
            row = jnp.where(li == ci, bc, row)
        par_ref[0] = row


def _norm_params(pc4):
    return pl.pallas_call(
        _stats_body,
        grid=(B, 2),
        in_specs=[pl.BlockSpec((1, C, SUBL, LANE), lambda b, p: (b, 0, 0, 0))],
        out_specs=pl.BlockSpec((1, 1, LANE), lambda b, p: (b, 0, 0)),
        out_shape=jax.ShapeDtypeStruct((B, 1, LANE), jnp.float32),
        scratch_shapes=[
            pltpu.SMEM((C,), jnp.float32),
        ],
    )(pc4)


ROWS = CH // LANE


def _sc_voxelize(pc4, params):
    mesh = plsc.VectorSubcoreMesh(core_axis_name="c", subcore_axis_name="s")

    @functools.partial(
        pl.kernel,
        out_type=jax.ShapeDtypeStruct((B, C, V), jnp.float32),
        mesh=mesh,
        compiler_params=pltpu.CompilerParams(
            use_tc_tiling_on_sc=False, needs_layout_passes=False
        ),
        scratch_types=[
            pltpu.VMEM((V,), jnp.float32),
            pltpu.VMEM((V,), jnp.float32),
            pltpu.VMEM((2, ROWS, LANE), jnp.float32),
            pltpu.VMEM((2, ROWS, LANE), jnp.float32),
            pltpu.VMEM((2, ROWS, LANE), jnp.float32),
            pltpu.VMEM((16,), jnp.float32),
            pltpu.VMEM((CH,), jnp.float32),
            pltpu.VMEM_SHARED((8, V), jnp.float32),
            pltpu.SemaphoreType.DMA,
            pltpu.SemaphoreType.DMA,
        ],
    )
    def k(pc_hbm, par_hbm, out_hbm, g0, g1, f0b, f1b, f2b, pb, cb, cnt_sh,
          sem0, sem1):
        c = lax.axis_index("c")
        s = lax.axis_index("s")
        bl = s % 8
        b = c * 8 + bl
        role = s // 8
        rolef = role.astype(jnp.float32)
        one_m_r = 1.0 - rolef

        pltpu.sync_copy(par_hbm.at[b, pl.ds(0, 16)], pb)
        pv = pb[...]
        iot = jnp.arange(16, dtype=jnp.int32)
        b0 = jnp.sum(jnp.where(iot == 0, pv, 0.0))
        b1 = jnp.sum(jnp.where(iot == 1, pv, 0.0))
        b2 = jnp.sum(jnp.where(iot == 2, pv, 0.0))
        av = jnp.sum(jnp.where(iot == 3, pv, 0.0))

        z = jnp.zeros((16,), jnp.float32)

        def zbody(i, _):
            base = i * 128
            for u in range(8):
                g0[pl.ds(base + u * 16, 16)] = z
                g1[pl.ds(base + u * 16, 16)] = z
            return 0

        lax.fori_loop(0, V // 128, zbody, 0)

        sems = [sem0, sem1]

        def start(kk, slot):
            r0 = kk * ROWS
            return [
                pltpu.async_copy(pc_hbm.at[b, 0, pl.ds(r0, ROWS), :], f0b.at[slot], sems[slot]),
                pltpu.async_copy(pc_hbm.at[b, 1, pl.ds(r0, ROWS), :], f1b.at[slot], sems[slot]),
                pltpu.async_copy(pc_hbm.at[b, 2, pl.ds(r0, ROWS), :], f2b.at[slot], sems[slot]),
            ]

        handles = [start(0, 0), start(1, 1)]
        for kk in range(NCHUNK):
            slot = kk & 1
            for h in handles[slot]:
                h.wait()

            def sbody(r, _):
                for u in range(8):
                    sl = pl.ds(u * 16, 16)
                    x0 = f0b[slot, r, sl]
                    x1 = f1b[slot, r, sl]
                    x2 = f2b[slot, r, sl]
                    q0 = jnp.clip(x0 * av + b0, 0.0, 31.0).astype(jnp.int32)
                    q1 = jnp.clip(x1 * av + b1, 0.0, 31.0).astype(jnp.int32)
                    q2 = jnp.clip(x2 * av + b2, 0.0, 31.0).astype(jnp.int32)
                    iv = (q0 << 10) | (q1 << 5) | q2
                    va = x0 + (x2 - x0) * rolef
                    vb = x1 * one_m_r + rolef
                    plsc.addupdate_scatter(g0, [iv], va)
                    plsc.addupdate_scatter(g1, [iv], vb)
                return 0

            lax.fori_loop(0, ROWS, sbody, 0)
            if kk + 2 < NCHUNK:
                handles[slot] = start(kk + 2, slot)

        @pl.when(role == 1)
        def _():
            pltpu.sync_copy(g1, cnt_sh.at[bl])

        plsc.subcore_barrier()

        def norm_block(kk, cnt_chunk_ref, targets):
            def mb(r, _):
                for u in range(8):
                    gsl = pl.ds(kk * CH + r * 128 + u * 16, 16)
                    cnt = cnt_chunk_ref[pl.ds(r * 128 + u * 16, 16)]
                    pos = cnt > 0.0
                    rec = jnp.where(pos, 1.0 / jnp.where(pos, cnt, 1.0), 0.0)
                    for g in targets:
                        g[gsl] = g[gsl] * rec
                return 0

            lax.fori_loop(0, ROWS, mb, 0)

        @pl.when(role == 0)
        def _():
            for kk in range(V // CH):
                pltpu.sync_copy(cnt_sh.at[bl, pl.ds(kk * CH, CH)], cb)
                norm_block(kk, cb, [g0, g1])
            pltpu.sync_copy(g0, out_hbm.at[b, 0])
            pltpu.sync_copy(g1, out_hbm.at[b, 1])

        @pl.when(role == 1)
        def _():
            def mb(r, _):
                for u in range(8):
                    sl = pl.ds(r * 128 + u * 16, 16)
                    cnt = g1[sl]
                    pos = cnt > 0.0
                    rec = jnp.where(pos, 1.0 / jnp.where(pos, cnt, 1.0), 0.0)
                    g0[sl] = g0[sl] * rec
                return 0

            lax.fori_loop(0, V // 128, mb, 0)
            pltpu.sync_copy(g0, out_hbm.at[b, 2])

    return k(pc4, params)


def _mm_body(x_ref, w_ref, b_ref, o_ref):
    o_ref[...] = (
        jnp.dot(x_ref[...], w_ref[...], preferred_element_type=jnp.float32)
        + b_ref[...]
    )


def _patch_matmul(pat, wt, bias):
    rows = B * G * G * G
    blk = 1024
    return pl.pallas_call(
        _mm_body,
        grid=(rows // blk,),
        in_specs=[
            pl.BlockSpec((blk, C * P * P * P), lambda i: (i, 0)),
            pl.BlockSpec((C * P * P * P, HIDDEN), lambda i: (0, 0)),
            pl.BlockSpec((1, HIDDEN), lambda i: (0, 0)),
        ],
        out_specs=pl.BlockSpec((blk, HIDDEN), lambda i: (i, 0)),
        out_shape=jax.ShapeDtypeStruct((rows, HIDDEN), jnp.float32),
    )(pat, wt, bias)


def kernel(point_cloud, W, b):
    pc4 = point_cloud.reshape(B, C, SUBL, LANE)
    params = _norm_params(pc4).reshape(B, LANE)
    avg = _sc_voxelize(pc4, params)
    pat = (
        avg.reshape(B, C, G, P, G, P, G, P)
        .transpose(0, 2, 4, 6, 1, 3, 5, 7)
        .reshape(B * G * G * G, C * P * P * P)
    )
    wt = W.reshape(HIDDEN, C * P * P * P).T
    tokens = _patch_matmul(pat, wt, b.reshape(1, HIDDEN))
    return tokens.reshape(B, G * G * G, HIDDEN)


# speedup vs baseline: 1.6408x; 1.6408x over previous
import functools

import jax
import jax.numpy as jnp
from jax import lax
from jax.experimental import pallas as pl
from jax.experimental.pallas import tpu as pltpu
from jax.experimental.pallas import tpu_sc as plsc

R = 32
P = 4
G = R // P
V = R * R * R
B = 16
C = 3
N = 131072
HIDDEN = 384
T = G * G * G
SUBL = 1024
LANE = 128
CH = 4096
NCHUNK = N // CH


def _stats_body(pc_ref, idx_ref, mean_ref, scale_ref):
    p = pl.program_id(1)

    @pl.when(p == 0)
    def _():
        for ci in range(C):
            mean_ref[ci] = jnp.sum(pc_ref[0, ci]) / N

    @pl.when(p == 1)
    def _():
        n0 = pc_ref[0, 0] - mean_ref[0]
        n1 = pc_ref[0, 1] - mean_ref[1]
        n2 = pc_ref[0, 2] - mean_ref[2]
        s = n0 * n0 + n1 * n1 + n2 * n2
        scale_ref[0] = jnp.sqrt(jnp.max(s)) * 2.0

    @pl.when(p == 2)
    def _():
        md2 = scale_ref[0]

        def quant(ci):
            nc = (pc_ref[0, ci] - mean_ref[ci]) / md2 + 0.5
            v = jnp.clip(nc * float(R), 0.0, float(R - 1))
            return v.astype(jnp.int32)

        idx_ref[0] = quant(0) * (R * R) + quant(1) * R + quant(2)


def _voxel_indices(pc4):
    return pl.pallas_call(
        _stats_body,
        grid=(B, 3),
        in_specs=[pl.BlockSpec((1, C, SUBL, LANE), lambda b, p: (b, 0, 0, 0))],
        out_specs=pl.BlockSpec((1, SUBL, LANE), lambda b, p: (b, 0, 0)),
        out_shape=jax.ShapeDtypeStruct((B, SUBL, LANE), jnp.int32),
        scratch_shapes=[
            pltpu.SMEM((C,), jnp.float32),
            pltpu.SMEM((1,), jnp.float32),
        ],
    )(pc4)


def _sc_voxelize(pcf, idxf):
    mesh = plsc.VectorSubcoreMesh(core_axis_name="c", subcore_axis_name="s")

    @functools.partial(
        pl.kernel,
        out_type=(
            jax.ShapeDtypeStruct((B * T, LANE), jnp.float32),
            jax.ShapeDtypeStruct((B * T, 64), jnp.float32),
        ),
        mesh=mesh,
        compiler_params=pltpu.CompilerParams(
            use_tc_tiling_on_sc=False, needs_layout_passes=False
        ),
        scratch_types=[
            pltpu.VMEM((V,), jnp.float32),
            pltpu.VMEM((V,), jnp.float32),
            pltpu.VMEM((2, CH // LANE, LANE), jnp.int32),
            pltpu.VMEM((2, CH // LANE, LANE), jnp.float32),
            pltpu.VMEM((2, CH // LANE, LANE), jnp.float32),
            pltpu.VMEM((T // 2, LANE), jnp.float32),
            pltpu.VMEM((CH,), jnp.float32),
            pltpu.VMEM_SHARED((8, 2, CH), jnp.float32),
            pltpu.SemaphoreType.DMA,
            pltpu.SemaphoreType.DMA,
        ],
    )
    def k(pc_hbm, idx_hbm, patA_hbm, patC_hbm, g0, g1, ib, f0b, f1b, pp, cb,
          cnt_sh, sem0, sem1):
        c = lax.axis_index("c")
        s = lax.axis_index("s")
        bl = s % 8
        b = c * 8 + bl
        role = s // 8
        rolef = role.astype(jnp.float32)
        one_m_r = 1.0 - rolef
        ch0 = role * 2
        ch1 = 1 + role

        z = jnp.zeros((16,), jnp.float32)

        def zbody(i, _):
            base = i * 128
            for u in range(8):
                g0[pl.ds(base + u * 16, 16)] = z
                g1[pl.ds(base + u * 16, 16)] = z
            return 0

        lax.fori_loop(0, V // 128, zbody, 0)

        sems = [sem0, sem1]

        def start(kk, slot):
            r0 = kk * (CH // LANE)
            return [
                pltpu.async_copy(idx_hbm.at[b, pl.ds(r0, CH // LANE), :], ib.at[slot], sems[slot]),
                pltpu.async_copy(pc_hbm.at[b, ch0, pl.ds(r0, CH // LANE), :], f0b.at[slot], sems[slot]),
                pltpu.async_copy(pc_hbm.at[b, ch1, pl.ds(r0, CH // LANE), :], f1b.at[slot], sems[slot]),
            ]

        handles = [start(0, 0), start(1, 1)]
        for kk in range(NCHUNK):
            slot = kk & 1
            for h in handles[slot]:
                h.wait()

            def sbody(r, _):
                for u in range(8):
                    sl = pl.ds(u * 16, 16)
                    iv = ib[slot, r, sl]
                    v0 = f0b[slot, r, sl]
                    v1 = f1b[slot, r, sl]
                    v1 = v1 * one_m_r + rolef
                    plsc.addupdate_scatter(g0, [iv], v0)
                    plsc.addupdate_scatter(g1, [iv], v1)
                return 0

            lax.fori_loop(0, CH // 128, sbody, 0)
            if kk + 2 < NCHUNK:
                handles[slot] = start(kk + 2, slot)

        lane_i = jnp.arange(16, dtype=jnp.int32)
        offs0 = (lane_i // 4) * 32 + (lane_i % 4)

        def recip(cnt):
            pos = cnt > 0.0
            return jnp.where(pos, 1.0 / jnp.where(pos, cnt, 1.0), 0.0)

        for gx in range(G):
            @pl.when(role == 1)
            def _():
                pltpu.sync_copy(g1.at[pl.ds(gx * CH, CH)], cnt_sh.at[bl, gx % 2])

            plsc.subcore_barrier()

            @pl.when(role == 0)
            def _():
                pltpu.sync_copy(cnt_sh.at[bl, gx % 2], cb)

                def tbody(tl, _):
                    row = (gx % 4) * 64 + tl
                    tbase = (tl // 8) * 128 + (tl % 8) * 4
                    for j in range(P):
                        rel = offs0 + (tbase + j * 1024)
                        absi = rel + gx * CH
                        rec = recip(plsc.load_gather(cb, [rel]))
                        pp[row, pl.ds(j * 16, 16)] = plsc.load_gather(g0, [absi]) * rec
                        pp[row, pl.ds(64 + j * 16, 16)] = plsc.load_gather(g1, [absi]) * rec
                    return 0

                lax.fori_loop(0, G * G, tbody, 0)
                if gx % 4 == 3:
                    dst_row = b * T + (gx // 4) * (T // 2)
                    pltpu.sync_copy(pp, patA_hbm.at[pl.ds(dst_row, T // 2), :])

            @pl.when(role == 1)
            def _():
                def tbody(tl, _):
                    row = (gx % 4) * 64 + tl
                    tbase = (tl // 8) * 128 + (tl % 8) * 4
                    for j in range(P):
                        absi = offs0 + (tbase + j * 1024 + gx * CH)
                        rec = recip(plsc.load_gather(g1, [absi]))
                        pp[row, pl.ds(j * 16, 16)] = plsc.load_gather(g0, [absi]) * rec
                    return 0

                lax.fori_loop(0, G * G, tbody, 0)
                if gx % 4 == 3:
                    dst_row = b * T + (gx // 4) * (T // 2)
                    pltpu.sync_copy(pp.at[:, pl.ds(0, 64)],
                                    patC_hbm.at[pl.ds(dst_row, T // 2), :])

    return k(pcf, idxf)


def _mm_body(a_ref, c_ref, w0_ref, w1_ref, b_ref, o_ref):
    acc = jnp.dot(a_ref[...], w0_ref[...], preferred_element_type=jnp.float32)
    acc += jnp.dot(c_ref[...], w1_ref[...], preferred_element_type=jnp.float32)
    o_ref[...] = acc + b_ref[...]


def _patch_matmul(patA, patC, w0, w1, bias):
    rows = B * T
    blk = 1024
    return pl.pallas_call(
        _mm_body,
        grid=(rows // blk,),
        in_specs=[
            pl.BlockSpec((blk, LANE), lambda i: (i, 0)),
            pl.BlockSpec((blk, 64), lambda i: (i, 0)),
            pl.BlockSpec((LANE, HIDDEN), lambda i: (0, 0)),
            pl.BlockSpec((64, HIDDEN), lambda i: (0, 0)),
            pl.BlockSpec((1, HIDDEN), lambda i: (0, 0)),
        ],
        out_specs=pl.BlockSpec((blk, HIDDEN), lambda i: (i, 0)),
        out_shape=jax.ShapeDtypeStruct((rows, HIDDEN), jnp.float32),
    )(patA, patC, w0, w1, bias)


def kernel(point_cloud, W, b):
    pc4 = point_cloud.reshape(B, C, SUBL, LANE)
    idx4 = _voxel_indices(pc4)
    patA, patC = _sc_voxelize(pc4, idx4)
    wt = W.reshape(HIDDEN, C * P * P * P).T
    w0 = wt[:LANE]
    w1 = wt[LANE:]
    tokens = _patch_matmul(patA, patC, w0, w1, b.reshape(1, HIDDEN))
    return tokens.reshape(B, T, HIDDEN)
